# baseline (speedup 1.0000x reference)
import jax
import jax.numpy as jnp
import numpy as np
from jax import lax
from jax.experimental import pallas as pl
from jax.experimental.pallas import tpu as pltpu
from jax.experimental.pallas import tpu_sc as plsc

NC, NS, L = 2, 16, 16
E = 256
TW = 300000
RW = 400000
TCH = 18752
RCH = 25000

_BASE = np.array([1, 0, 0, 0, 0, 1, 0, 0, 0, 0, 1, 0, 0, 0, 0, 1], np.float32)
_C1 = np.array([-2, 2, 2, 0, 2, -2, 2, 0, 2, 2, -2, 0, 0, 0, 0, 0], np.float32)
_C2 = np.array([-2, -2, 2, 0, 2, -2, -2, 0, -2, 2, -2, 0, 0, 0, 0, 0], np.float32)
_TCOEF = np.array([0, 0, 0, 1, 0, 0, 0, 1, 0, 0, 0, 1, 0, 0, 0, 0], np.float32)
_A = np.array([2, 1, 0, 0, 1, 1, 2, 0, 1, 0, 1, 0, 0, 0, 0, 0], np.int32)
_B = np.array([2, 2, 2, 0, 2, 1, 3, 0, 3, 1, 1, 0, 0, 0, 0, 0], np.int32)
_C = np.array([3, 0, 1, 0, 0, 3, 0, 0, 0, 2, 2, 0, 0, 0, 0, 0], np.int32)
_D = np.array([3, 3, 3, 0, 3, 3, 1, 0, 2, 3, 2, 0, 0, 0, 0, 0], np.int32)
_TIDX = np.array([0, 0, 0, 0, 0, 0, 0, 1, 0, 0, 0, 2, 0, 0, 0, 0], np.int32)
_PAT3 = (np.arange(16) % 3).astype(np.int32)
_PAT4 = (np.arange(16) % 4).astype(np.int32)
_CI = np.stack([_A, _B, _C, _D, _TIDX, _PAT3, _PAT4, np.zeros(16, np.int32)])
_CF = np.stack([_BASE, _C1, _C2, _TCOEF])


def _body(idx16, embt, w1t, b1t, w2t, b2t, w3t, b3t,
          embr, w1r, b1r, w2r, b2r, w3r, b3r,
          tb, rb, ci, cf,
          c2w_o, tbn_o, rbn_o,
          idx_v, ci_v, cf_v, e_v, w_v, w3_v, bs_v, hs_v, hf_v, pv_v,
          t_v, r_v, sidx_v, sval_v, chunk_v,
          sh256, shpart,
          sem_in, sem_out):
  cid = lax.axis_index("c")
  sid = lax.axis_index("s")

  t_off = jnp.minimum(sid * TCH, TW - TCH)
  r_off = sid * RCH
  t_in = pltpu.make_async_copy(tb.at[pl.ds(t_off, TCH)],
                               chunk_v.at[pl.ds(0, TCH)], sem_in)
  r_in = pltpu.make_async_copy(rb.at[pl.ds(r_off, RCH)],
                               chunk_v.at[pl.ds(0, RCH)], sem_in)

  @pl.when(cid == 0)
  def _():
    t_in.start()

  @pl.when(cid == 1)
  def _():
    r_in.start()

  pltpu.sync_copy(idx16, idx_v)
  pltpu.sync_copy(ci, ci_v)
  pltpu.sync_copy(cf, cf_v)

  def run_net(emb_h, w1_h, b1_h, w2_h, b2_h, w3_h, b3_h, out_v):
    pltpu.sync_copy(emb_h.at[idx_v], e_v)
    pltpu.sync_copy(w1_h.at[:, pl.ds(sid * L, L)], w_v)
    pltpu.sync_copy(b1_h.at[pl.ds(sid * L, L)], bs_v)

    def step1(i, acc):
      si = plsc.load_gather(
          e_v, [jnp.zeros((L,), jnp.int32), jnp.full((L,), i, jnp.int32)])
      return acc + si * w_v[i]

    h1 = jnp.maximum(lax.fori_loop(0, E, step1, bs_v[...]), 0.0)
    hs_v[...] = h1
    pltpu.sync_copy(hs_v, sh256.at[pl.ds(sid * L, L)])
    plsc.subcore_barrier()
    pltpu.sync_copy(sh256, hf_v)
    plsc.subcore_barrier()

    pltpu.sync_copy(w2_h.at[:, pl.ds(sid * L, L)], w_v)
    pltpu.sync_copy(b2_h.at[pl.ds(sid * L, L)], bs_v)

    def step2(i, acc):
      si = plsc.load_gather(hf_v, [jnp.full((L,), i, jnp.int32)])
      return acc + si * w_v[i]

    h2 = jnp.maximum(lax.fori_loop(0, E, step2, bs_v[...]), 0.0)
    hs_v[...] = h2
    pltpu.sync_copy(hs_v, sh256.at[pl.ds(sid * L, L)])
    plsc.subcore_barrier()
    pltpu.sync_copy(sh256, hf_v)
    plsc.subcore_barrier()

    pltpu.sync_copy(w3_h.at[pl.ds(sid * L, L), :], w3_v)

    def step3(k, acc):
      si = plsc.load_gather(hf_v, [jnp.full((L,), sid * L + k, jnp.int32)])
      return acc + si * w3_v[k]

    part = lax.fori_loop(0, L, step3, jnp.zeros((L,), jnp.float32))
    hs_v[...] = part
    pltpu.sync_copy(hs_v, shpart.at[sid])
    plsc.subcore_barrier()
    pltpu.sync_copy(shpart, pv_v)
    plsc.subcore_barrier()
    pltpu.sync_copy(b3_h, bs_v)

    def stepr(k, acc):
      return acc + pv_v[k]

    out_v[...] = lax.fori_loop(0, NS, stepr, bs_v[...])

  run_net(embt, w1t, b1t, w2t, b2t, w3t, b3t, t_v)
  run_net(embr, w1r, b1r, w2r, b2r, w3r, b3r, r_v)

  rr = r_v[...]
  lane = lax.iota(jnp.int32, L)
  s = jnp.sum(jnp.where(lane < 4, rr * rr, 0.0))
  sv = jnp.full((L,), s)
  sv_safe = jnp.maximum(sv, 1e-37)
  bits = lax.bitcast_convert_type(sv_safe, jnp.int32)
  y = lax.bitcast_convert_type(
      jnp.full((L,), 0x5F3759DF, jnp.int32) - (bits >> 1), jnp.float32)
  y = y * (1.5 - 0.5 * sv_safe * y * y)
  y = y * (1.5 - 0.5 * sv_safe * y * y)
  y = y * (1.5 - 0.5 * sv_safe * y * y)
  y = y * (1.5 - 0.5 * sv_safe * y * y)
  norm = sv * y
  rq = rr * (1.0 / (norm + 1e-8))
  r_v[...] = rq

  ra = plsc.load_gather(r_v, [ci_v[0]])
  rb_q = plsc.load_gather(r_v, [ci_v[1]])
  rc = plsc.load_gather(r_v, [ci_v[2]])
  rd = plsc.load_gather(r_v, [ci_v[3]])
  tg = plsc.load_gather(t_v, [ci_v[4]])
  hs_v[...] = (cf_v[0] + cf_v[1] * ra * rb_q + cf_v[2] * rc * rd
               + cf_v[3] * tg)

  t_out = pltpu.make_async_copy(chunk_v.at[pl.ds(0, TCH)],
                                tbn_o.at[pl.ds(t_off, TCH)], sem_out)
  r_out = pltpu.make_async_copy(chunk_v.at[pl.ds(0, RCH)],
                                rbn_o.at[pl.ds(r_off, RCH)], sem_out)

  @pl.when(cid == 0)
  def _():
    t_in.wait()
    t_out.start()
    t_out.wait()

  @pl.when(cid == 1)
  def _():
    r_in.wait()
    r_out.start()
    r_out.wait()

  plsc.subcore_barrier()

  @pl.when((cid == 0) & (sid == 0))
  def _():
    pltpu.sync_copy(hs_v, c2w_o)
    sidx_v[...] = idx_v[...] * 3 + ci_v[5]
    sval_v[...] = plsc.load_gather(t_v, [ci_v[5]])
    pltpu.sync_copy(sval_v, tbn_o.at[sidx_v])

  @pl.when((cid == 1) & (sid == 0))
  def _():
    sidx_v[...] = idx_v[...] * 4 + ci_v[6]
    sval_v[...] = plsc.load_gather(r_v, [ci_v[6]])
    pltpu.sync_copy(sval_v, rbn_o.at[sidx_v])


def kernel(cam_id, emb_t, W1_t, b1_t, W2_t, b2_t, W3_t, b3_t,
           emb_r, W1_r, b1_r, W2_r, b2_r, W3_r, b3_r, t_buf, r_buf):
  n = t_buf.shape[0]
  idx = jnp.asarray(cam_id, jnp.int32) - 1
  idx16 = jnp.full((16,), idx, jnp.int32)
  w3tp = jnp.pad(W3_t, ((0, 0), (0, 16 - W3_t.shape[1])))
  b3tp = jnp.pad(b3_t, (0, 16 - b3_t.shape[0]))
  w3rp = jnp.pad(W3_r, ((0, 0), (0, 16 - W3_r.shape[1])))
  b3rp = jnp.pad(b3_r, (0, 16 - b3_r.shape[0]))
  ci = jnp.asarray(_CI)
  cf = jnp.asarray(_CF)
  tb = t_buf.reshape(-1)
  rb = r_buf.reshape(-1)

  mesh = plsc.VectorSubcoreMesh(core_axis_name="c", subcore_axis_name="s")
  f = pl.kernel(
      _body,
      out_type=(
          jax.ShapeDtypeStruct((16,), jnp.float32),
          jax.ShapeDtypeStruct((TW,), jnp.float32),
          jax.ShapeDtypeStruct((RW,), jnp.float32),
      ),
      mesh=mesh,
      compiler_params=pltpu.CompilerParams(use_tc_tiling_on_sc=False,
                                           needs_layout_passes=False),
      scratch_types=[
          pltpu.VMEM((16,), jnp.int32),
          pltpu.VMEM((8, 16), jnp.int32),
          pltpu.VMEM((4, 16), jnp.float32),
          pltpu.VMEM((16, E), jnp.float32),
          pltpu.VMEM((E, L), jnp.float32),
          pltpu.VMEM((L, L), jnp.float32),
          pltpu.VMEM((L,), jnp.float32),
          pltpu.VMEM((L,), jnp.float32),
          pltpu.VMEM((E,), jnp.float32),
          pltpu.VMEM((NS, L), jnp.float32),
          pltpu.VMEM((L,), jnp.float32),
          pltpu.VMEM((L,), jnp.float32),
          pltpu.VMEM((16,), jnp.int32),
          pltpu.VMEM((16,), jnp.float32),
          pltpu.VMEM((RCH,), jnp.float32),
          pltpu.VMEM_SHARED((E,), jnp.float32),
          pltpu.VMEM_SHARED((NS, L), jnp.float32),
          pltpu.SemaphoreType.DMA,
          pltpu.SemaphoreType.DMA,
      ],
  )
  c2w16, tbn, rbn = f(idx16, emb_t, W1_t, b1_t, W2_t, b2_t, w3tp, b3tp,
                      emb_r, W1_r, b1_r, W2_r, b2_r, w3rp, b3rp,
                      tb, rb, ci, cf)
  return (c2w16.reshape(4, 4), tbn.reshape(n, 3), rbn.reshape(n, 4))


# speedup vs baseline: 1.0122x; 1.0122x over previous
import jax
import jax.numpy as jnp
import numpy as np
from jax import lax
from jax.experimental import pallas as pl
from jax.experimental.pallas import tpu as pltpu
from jax.experimental.pallas import tpu_sc as plsc

NC, NS, L = 2, 16, 16
E = 256
TW = 300000
RW = 400000
TCH = 18752
RCH = 25000

_BASE = np.array([1, 0, 0, 0, 0, 1, 0, 0, 0, 0, 1, 0, 0, 0, 0, 1], np.float32)
_C1 = np.array([-2, 2, 2, 0, 2, -2, 2, 0, 2, 2, -2, 0, 0, 0, 0, 0], np.float32)
_C2 = np.array([-2, -2, 2, 0, 2, -2, -2, 0, -2, 2, -2, 0, 0, 0, 0, 0], np.float32)
_TCOEF = np.array([0, 0, 0, 1, 0, 0, 0, 1, 0, 0, 0, 1, 0, 0, 0, 0], np.float32)
_A = np.array([2, 1, 0, 0, 1, 1, 2, 0, 1, 0, 1, 0, 0, 0, 0, 0], np.int32)
_B = np.array([2, 2, 2, 0, 2, 1, 3, 0, 3, 1, 1, 0, 0, 0, 0, 0], np.int32)
_C = np.array([3, 0, 1, 0, 0, 3, 0, 0, 0, 2, 2, 0, 0, 0, 0, 0], np.int32)
_D = np.array([3, 3, 3, 0, 3, 3, 1, 0, 2, 3, 2, 0, 0, 0, 0, 0], np.int32)
_TIDX = np.array([0, 0, 0, 0, 0, 0, 0, 1, 0, 0, 0, 2, 0, 0, 0, 0], np.int32)
_PAT3 = (np.arange(16) % 3).astype(np.int32)
_PAT4 = (np.arange(16) % 4).astype(np.int32)
_CI = np.stack([_A, _B, _C, _D, _TIDX, _PAT3, _PAT4,
                np.arange(16, dtype=np.int32)])
_CF = np.stack([_BASE, _C1, _C2, _TCOEF])
_IIDX = np.arange(256, dtype=np.int32).reshape(2, 128)


def _body(idx16, embt, w1t, b1t, w2t, b2t, w3t, b3t,
          embr, w1r, b1r, w2r, b2r, w3r, b3r,
          tb, rb, ci, cf, iidx,
          c2w_o, tbn_o, rbn_o,
          idx_v, ci_v, cf_v, ii_v, e_v, w1_v, w2_v, w3_v,
          part_v, p3_v, hc_v, bias_v, t_v, r_v, sidx_v, sval_v, chunk_v,
          shA_t, shB_t, sh3_t, shA_r, shB_r, sh3_r,
          sem_in, sem_out):
  cid = lax.axis_index("c")
  sid = lax.axis_index("s")
  z16 = jnp.zeros((L,), jnp.int32)
  lane = lax.iota(jnp.int32, L)

  t_off = jnp.minimum(sid * TCH, TW - TCH)
  r_off = sid * RCH
  t_in = pltpu.make_async_copy(tb.at[pl.ds(t_off, TCH)],
                               chunk_v.at[pl.ds(0, TCH)], sem_in)
  r_in = pltpu.make_async_copy(rb.at[pl.ds(r_off, RCH)],
                               chunk_v.at[pl.ds(0, RCH)], sem_in)

  @pl.when(cid == 0)
  def _():
    t_in.start()

  @pl.when(cid == 1)
  def _():
    r_in.start()

  pltpu.sync_copy(idx16, idx_v)
  pltpu.sync_copy(ci, ci_v)
  pltpu.sync_copy(cf, cf_v)
  pltpu.sync_copy(iidx, ii_v)

  @pl.when(sid == 0)
  def _():
    for b_h, sh in ((b1t, shA_t), (b2t, shB_t), (b1r, shA_r), (b2r, shB_r)):
      pltpu.sync_copy(b_h, bias_v)
      pltpu.sync_copy(bias_v, sh)
    for b_h, sh in ((b3t, sh3_t), (b3r, sh3_r)):
      pltpu.sync_copy(b_h, p3_v)
      pltpu.sync_copy(p3_v, sh)

  def bcast(src_ref, base, k, relu):
    v = plsc.load_gather(src_ref, [jnp.full((L,), base + k, jnp.int32)])
    return jnp.maximum(v, 0.0) if relu else v

  def layer_big(src_ref, base, relu, w_ref, sh_acc):
    svs = [bcast(src_ref, base, k, relu) for k in range(L)]

    def jc_body(jc, carry):
      acc = None
      for k in range(L):
        wk = plsc.load_gather(w_ref, [jnp.full((L,), k, jnp.int32),
                                      jc * L + lane])
        acc = svs[k] * wk if acc is None else acc + svs[k] * wk
      part_v[pl.ds(jc * L, L)] = acc
      return carry

    lax.fori_loop(0, L, jc_body, 0, unroll=2)
    pltpu.sync_copy(part_v.at[pl.ds(0, 128)], sh_acc.at[ii_v.at[0]],
                    add=True)
    pltpu.sync_copy(part_v.at[pl.ds(128, 128)], sh_acc.at[ii_v.at[1]],
                    add=True)

  def layer_small(src_ref, base, w_ref, sh_acc):
    svs = [bcast(src_ref, base, k, True) for k in range(L)]
    acc = None
    for k in range(L):
      wk = plsc.load_gather(w_ref, [jnp.full((L,), k, jnp.int32), lane])
      acc = svs[k] * wk if acc is None else acc + svs[k] * wk
    p3_v[...] = acc
    pltpu.sync_copy(p3_v, sh_acc.at[ci_v.at[7]], add=True)

  pltpu.sync_copy(embt.at[idx_v], e_v)
  pltpu.sync_copy(w1t.at[pl.ds(sid * L, L), :], w1_v)
  pltpu.sync_copy(w2t.at[pl.ds(sid * L, L), :], w2_v)
  pltpu.sync_copy(w3t.at[pl.ds(sid * L, L), :], w3_v)
  plsc.subcore_barrier()

  hc_v[pl.ds(L, L)] = plsc.load_gather(e_v, [z16, sid * L + lane])
  layer_big(hc_v, L, False, w1_v, shA_t)
  plsc.subcore_barrier()
  pltpu.sync_copy(shA_t.at[pl.ds(sid * L, L)], hc_v.at[pl.ds(L, L)])
  layer_big(hc_v, L, True, w2_v, shB_t)
  plsc.subcore_barrier()
  pltpu.sync_copy(shB_t.at[pl.ds(sid * L, L)], hc_v.at[pl.ds(L, L)])
  layer_small(hc_v, L, w3_v, sh3_t)

  pltpu.sync_copy(embr.at[idx_v], e_v)
  pltpu.sync_copy(w1r.at[pl.ds(sid * L, L), :], w1_v)
  pltpu.sync_copy(w2r.at[pl.ds(sid * L, L), :], w2_v)
  pltpu.sync_copy(w3r.at[pl.ds(sid * L, L), :], w3_v)

  hc_v[pl.ds(L, L)] = plsc.load_gather(e_v, [z16, sid * L + lane])
  layer_big(hc_v, L, False, w1_v, shA_r)
  plsc.subcore_barrier()
  pltpu.sync_copy(shA_r.at[pl.ds(sid * L, L)], hc_v.at[pl.ds(L, L)])
  layer_big(hc_v, L, True, w2_v, shB_r)
  plsc.subcore_barrier()
  pltpu.sync_copy(shB_r.at[pl.ds(sid * L, L)], hc_v.at[pl.ds(L, L)])
  layer_small(hc_v, L, w3_v, sh3_r)
  plsc.subcore_barrier()

  pltpu.sync_copy(sh3_t, t_v)
  pltpu.sync_copy(sh3_r, r_v)

  rr = r_v[...]
  s = jnp.sum(jnp.where(lane < 4, rr * rr, 0.0))
  sv = jnp.full((L,), s)
  sv_safe = jnp.maximum(sv, 1e-37)
  bits = lax.bitcast_convert_type(sv_safe, jnp.int32)
  y = lax.bitcast_convert_type(
      jnp.full((L,), 0x5F3759DF, jnp.int32) - (bits >> 1), jnp.float32)
  y = y * (1.5 - 0.5 * sv_safe * y * y)
  y = y * (1.5 - 0.5 * sv_safe * y * y)
  y = y * (1.5 - 0.5 * sv_safe * y * y)
  y = y * (1.5 - 0.5 * sv_safe * y * y)
  norm = sv * y
  rq = rr * (1.0 / (norm + 1e-8))
  r_v[...] = rq

  ra = plsc.load_gather(r_v, [ci_v[0]])
  rb_q = plsc.load_gather(r_v, [ci_v[1]])
  rc = plsc.load_gather(r_v, [ci_v[2]])
  rd = plsc.load_gather(r_v, [ci_v[3]])
  tg = plsc.load_gather(t_v, [ci_v[4]])
  bias_v[pl.ds(0, L)] = (cf_v[0] + cf_v[1] * ra * rb_q + cf_v[2] * rc * rd
                         + cf_v[3] * tg)

  t_out = pltpu.make_async_copy(chunk_v.at[pl.ds(0, TCH)],
                                tbn_o.at[pl.ds(t_off, TCH)], sem_out)
  r_out = pltpu.make_async_copy(chunk_v.at[pl.ds(0, RCH)],
                                rbn_o.at[pl.ds(r_off, RCH)], sem_out)

  @pl.when(cid == 0)
  def _():
    t_in.wait()
    t_out.start()
    t_out.wait()

  @pl.when(cid == 1)
  def _():
    r_in.wait()
    r_out.start()
    r_out.wait()

  plsc.subcore_barrier()

  @pl.when((cid == 0) & (sid == 0))
  def _():
    pltpu.sync_copy(bias_v.at[pl.ds(0, L)], c2w_o)
    sidx_v[...] = idx_v[...] * 3 + ci_v[5]
    sval_v[...] = plsc.load_gather(t_v, [ci_v[5]])
    pltpu.sync_copy(sval_v, tbn_o.at[sidx_v])

  @pl.when((cid == 1) & (sid == 0))
  def _():
    sidx_v[...] = idx_v[...] * 4 + ci_v[6]
    sval_v[...] = plsc.load_gather(r_v, [ci_v[6]])
    pltpu.sync_copy(sval_v, rbn_o.at[sidx_v])


def kernel(cam_id, emb_t, W1_t, b1_t, W2_t, b2_t, W3_t, b3_t,
           emb_r, W1_r, b1_r, W2_r, b2_r, W3_r, b3_r, t_buf, r_buf):
  n = t_buf.shape[0]
  idx = jnp.asarray(cam_id, jnp.int32) - 1
  idx16 = jnp.full((16,), idx, jnp.int32)
  w3tp = jnp.pad(W3_t, ((0, 0), (0, 16 - W3_t.shape[1])))
  b3tp = jnp.pad(b3_t, (0, 16 - b3_t.shape[0]))
  w3rp = jnp.pad(W3_r, ((0, 0), (0, 16 - W3_r.shape[1])))
  b3rp = jnp.pad(b3_r, (0, 16 - b3_r.shape[0]))
  ci = jnp.asarray(_CI)
  cf = jnp.asarray(_CF)
  iidx = jnp.asarray(_IIDX)
  tb = t_buf.reshape(-1)
  rb = r_buf.reshape(-1)

  mesh = plsc.VectorSubcoreMesh(core_axis_name="c", subcore_axis_name="s")
  f = pl.kernel(
      _body,
      out_type=(
          jax.ShapeDtypeStruct((16,), jnp.float32),
          jax.ShapeDtypeStruct((TW,), jnp.float32),
          jax.ShapeDtypeStruct((RW,), jnp.float32),
      ),
      mesh=mesh,
      compiler_params=pltpu.CompilerParams(use_tc_tiling_on_sc=False,
                                           needs_layout_passes=False),
      scratch_types=[
          pltpu.VMEM((16,), jnp.int32),
          pltpu.VMEM((8, 16), jnp.int32),
          pltpu.VMEM((4, 16), jnp.float32),
          pltpu.VMEM((2, 128), jnp.int32),
          pltpu.VMEM((16, E), jnp.float32),
          pltpu.VMEM((L, E), jnp.float32),
          pltpu.VMEM((L, E), jnp.float32),
          pltpu.VMEM((L, L), jnp.float32),
          pltpu.VMEM((E,), jnp.float32),
          pltpu.VMEM((L,), jnp.float32),
          pltpu.VMEM((2 * L,), jnp.float32),
          pltpu.VMEM((E,), jnp.float32),
          pltpu.VMEM((L,), jnp.float32),
          pltpu.VMEM((L,), jnp.float32),
          pltpu.VMEM((16,), jnp.int32),
          pltpu.VMEM((16,), jnp.float32),
          pltpu.VMEM((RCH,), jnp.float32),
          pltpu.VMEM_SHARED((E,), jnp.float32),
          pltpu.VMEM_SHARED((E,), jnp.float32),
          pltpu.VMEM_SHARED((L,), jnp.float32),
          pltpu.VMEM_SHARED((E,), jnp.float32),
          pltpu.VMEM_SHARED((E,), jnp.float32),
          pltpu.VMEM_SHARED((L,), jnp.float32),
          pltpu.SemaphoreType.DMA,
          pltpu.SemaphoreType.DMA,
      ],
  )
  c2w16, tbn, rbn = f(idx16, emb_t, W1_t, b1_t, W2_t, b2_t, w3tp, b3tp,
                      emb_r, W1_r, b1_r, W2_r, b2_r, w3rp, b3rp,
                      tb, rb, ci, cf, iidx)
  return (c2w16.reshape(4, 4), tbn.reshape(n, 3), rbn.reshape(n, 4))


# speedup vs baseline: 1.5651x; 1.5462x over previous
import jax
import jax.numpy as jnp
import numpy as np
from jax import lax
from jax.experimental import pallas as pl
from jax.experimental.pallas import tpu as pltpu
from jax.experimental.pallas import tpu_sc as plsc

NC, NS, L = 2, 16, 16
E = 256
TW = 300000
RW = 400000
TCH = 18752
RCH = 25000

_BASE = np.array([1, 0, 0, 0, 0, 1, 0, 0, 0, 0, 1, 0, 0, 0, 0, 1], np.float32)
_C1 = np.array([-2, 2, 2, 0, 2, -2, 2, 0, 2, 2, -2, 0, 0, 0, 0, 0], np.float32)
_C2 = np.array([-2, -2, 2, 0, 2, -2, -2, 0, -2, 2, -2, 0, 0, 0, 0, 0], np.float32)
_TCOEF = np.array([0, 0, 0, 1, 0, 0, 0, 1, 0, 0, 0, 1, 0, 0, 0, 0], np.float32)
_A = np.array([2, 1, 0, 0, 1, 1, 2, 0, 1, 0, 1, 0, 0, 0, 0, 0], np.int32)
_B = np.array([2, 2, 2, 0, 2, 1, 3, 0, 3, 1, 1, 0, 0, 0, 0, 0], np.int32)
_C = np.array([3, 0, 1, 0, 0, 3, 0, 0, 0, 2, 2, 0, 0, 0, 0, 0], np.int32)
_D = np.array([3, 3, 3, 0, 3, 3, 1, 0, 2, 3, 2, 0, 0, 0, 0, 0], np.int32)
_TIDX = np.array([0, 0, 0, 0, 0, 0, 0, 1, 0, 0, 0, 2, 0, 0, 0, 0], np.int32)
_PAT3 = (np.arange(16) % 3).astype(np.int32)
_PAT4 = (np.arange(16) % 4).astype(np.int32)
_CI = np.stack([_A, _B, _C, _D, _TIDX, _PAT3, _PAT4,
                np.arange(16, dtype=np.int32)])
_CF = np.stack([_BASE, _C1, _C2, _TCOEF])
_IIDX = np.arange(256, dtype=np.int32).reshape(2, 128)


def _body(idx16, embt, w1t, b1t, w2t, b2t, w3t, b3t,
          embr, w1r, b1r, w2r, b2r, w3r, b3r,
          tb, rb, ci, cf, iidx,
          c2w_o, tbn_o, rbn_o,
          idx_v, ci_v, cf_v, ii_v, e_v, w1_v, w2_v, w3_v,
          part_v, p3_v, hc_v, bias_v, t_v, r_v, sidx_v, sval_v, chunk_v,
          shA_t, shB_t, sh3_t, shA_r, shB_r, sh3_r,
          sem_in, sem_out):
  cid = lax.axis_index("c")
  sid = lax.axis_index("s")
  z16 = jnp.zeros((L,), jnp.int32)
  lane = lax.iota(jnp.int32, L)

  t_off = jnp.minimum(sid * TCH, TW - TCH)
  r_off = sid * RCH
  t_in = pltpu.make_async_copy(tb.at[pl.ds(t_off, TCH)],
                               chunk_v.at[pl.ds(0, TCH)], sem_in)
  r_in = pltpu.make_async_copy(rb.at[pl.ds(r_off, RCH)],
                               chunk_v.at[pl.ds(0, RCH)], sem_in)

  @pl.when(cid == 0)
  def _():
    t_in.start()

  @pl.when(cid == 1)
  def _():
    r_in.start()

  pltpu.sync_copy(idx16, idx_v)
  pltpu.sync_copy(ci, ci_v)
  pltpu.sync_copy(cf, cf_v)
  pltpu.sync_copy(iidx, ii_v)

  @pl.when(sid == 0)
  def _():
    for b_h, sh in ((b1t, shA_t), (b2t, shB_t), (b1r, shA_r), (b2r, shB_r)):
      pltpu.sync_copy(b_h, bias_v)
      pltpu.sync_copy(bias_v, sh)
    for b_h, sh in ((b3t, sh3_t), (b3r, sh3_r)):
      pltpu.sync_copy(b_h, p3_v)
      pltpu.sync_copy(p3_v, sh)

  def bcast(src_ref, base, k, relu):
    v = plsc.load_gather(src_ref, [jnp.full((L,), base + k, jnp.int32)])
    return jnp.maximum(v, 0.0) if relu else v

  def layer_big(src_ref, base, relu, w_ref, sh_acc):
    svs = [bcast(src_ref, base, k, relu) for k in range(L)]

    def jc_body(jc, carry):
      acc = None
      for k in range(L):
        wk = plsc.load_gather(w_ref, [jnp.full((L,), k, jnp.int32),
                                      jc * L + lane])
        acc = svs[k] * wk if acc is None else acc + svs[k] * wk
      part_v[pl.ds(jc * L, L)] = acc
      return carry

    lax.fori_loop(0, L, jc_body, 0, unroll=2)
    pltpu.sync_copy(part_v.at[pl.ds(0, 128)], sh_acc.at[ii_v.at[0]],
                    add=True)
    pltpu.sync_copy(part_v.at[pl.ds(128, 128)], sh_acc.at[ii_v.at[1]],
                    add=True)

  def layer_small(src_ref, base, w_ref, sh_acc):
    svs = [bcast(src_ref, base, k, True) for k in range(L)]
    acc = None
    for k in range(L):
      wk = plsc.load_gather(w_ref, [jnp.full((L,), k, jnp.int32), lane])
      acc = svs[k] * wk if acc is None else acc + svs[k] * wk
    p3_v[...] = acc
    pltpu.sync_copy(p3_v, sh_acc.at[ci_v.at[7]], add=True)

  pltpu.sync_copy(embt.at[idx_v], e_v)
  pltpu.sync_copy(w1t.at[pl.ds(sid * L, L), :], w1_v)
  pltpu.sync_copy(w2t.at[pl.ds(sid * L, L), :], w2_v)
  pltpu.sync_copy(w3t.at[pl.ds(sid * L, L), :], w3_v)
  plsc.subcore_barrier()

  hc_v[pl.ds(L, L)] = plsc.load_gather(e_v, [z16, sid * L + lane])
  layer_big(hc_v, L, False, w1_v, shA_t)
  plsc.subcore_barrier()
  pltpu.sync_copy(shA_t.at[pl.ds(sid * L, L)], hc_v.at[pl.ds(L, L)])
  layer_big(hc_v, L, True, w2_v, shB_t)
  plsc.subcore_barrier()
  pltpu.sync_copy(shB_t.at[pl.ds(sid * L, L)], hc_v.at[pl.ds(L, L)])
  layer_small(hc_v, L, w3_v, sh3_t)

  pltpu.sync_copy(embr.at[idx_v], e_v)
  pltpu.sync_copy(w1r.at[pl.ds(sid * L, L), :], w1_v)
  pltpu.sync_copy(w2r.at[pl.ds(sid * L, L), :], w2_v)
  pltpu.sync_copy(w3r.at[pl.ds(sid * L, L), :], w3_v)

  hc_v[pl.ds(L, L)] = plsc.load_gather(e_v, [z16, sid * L + lane])
  layer_big(hc_v, L, False, w1_v, shA_r)
  plsc.subcore_barrier()
  pltpu.sync_copy(shA_r.at[pl.ds(sid * L, L)], hc_v.at[pl.ds(L, L)])
  layer_big(hc_v, L, True, w2_v, shB_r)
  plsc.subcore_barrier()
  pltpu.sync_copy(shB_r.at[pl.ds(sid * L, L)], hc_v.at[pl.ds(L, L)])
  layer_small(hc_v, L, w3_v, sh3_r)
  plsc.subcore_barrier()

  pltpu.sync_copy(sh3_t, t_v)
  pltpu.sync_copy(sh3_r, r_v)

  rr = r_v[...]
  s = jnp.sum(jnp.where(lane < 4, rr * rr, 0.0))
  sv = jnp.full((L,), s)
  sv_safe = jnp.maximum(sv, 1e-37)
  bits = lax.bitcast_convert_type(sv_safe, jnp.int32)
  y = lax.bitcast_convert_type(
      jnp.full((L,), 0x5F3759DF, jnp.int32) - (bits >> 1), jnp.float32)
  y = y * (1.5 - 0.5 * sv_safe * y * y)
  y = y * (1.5 - 0.5 * sv_safe * y * y)
  y = y * (1.5 - 0.5 * sv_safe * y * y)
  y = y * (1.5 - 0.5 * sv_safe * y * y)
  norm = sv * y
  rq = rr * (1.0 / (norm + 1e-8))
  r_v[...] = rq

  ra = plsc.load_gather(r_v, [ci_v[0]])
  rb_q = plsc.load_gather(r_v, [ci_v[1]])
  rc = plsc.load_gather(r_v, [ci_v[2]])
  rd = plsc.load_gather(r_v, [ci_v[3]])
  tg = plsc.load_gather(t_v, [ci_v[4]])
  bias_v[pl.ds(0, L)] = (cf_v[0] + cf_v[1] * ra * rb_q + cf_v[2] * rc * rd
                         + cf_v[3] * tg)

  t_out = pltpu.make_async_copy(chunk_v.at[pl.ds(0, TCH)],
                                tbn_o.at[pl.ds(t_off, TCH)], sem_out)
  r_out = pltpu.make_async_copy(chunk_v.at[pl.ds(0, RCH)],
                                rbn_o.at[pl.ds(r_off, RCH)], sem_out)

  @pl.when(cid == 0)
  def _():
    t_in.wait()
    t_out.start()
    t_out.wait()

  @pl.when(cid == 1)
  def _():
    r_in.wait()
    r_out.start()
    r_out.wait()

  plsc.subcore_barrier()

  @pl.when((cid == 0) & (sid == 0))
  def _():
    pltpu.sync_copy(bias_v.at[pl.ds(0, L)], c2w_o)
    sidx_v[...] = idx_v[...] * 3 + ci_v[5]
    sval_v[...] = plsc.load_gather(t_v, [ci_v[5]])
    pltpu.sync_copy(sval_v, tbn_o.at[sidx_v])

  @pl.when((cid == 1) & (sid == 0))
  def _():
    sidx_v[...] = idx_v[...] * 4 + ci_v[6]
    sval_v[...] = plsc.load_gather(r_v, [ci_v[6]])
    pltpu.sync_copy(sval_v, rbn_o.at[sidx_v])


def kernel(cam_id, emb_t, W1_t, b1_t, W2_t, b2_t, W3_t, b3_t,
           emb_r, W1_r, b1_r, W2_r, b2_r, W3_r, b3_r, t_buf, r_buf):
  n = t_buf.shape[0]
  idx = jnp.asarray(cam_id, jnp.int32) - 1
  idx16 = jnp.full((16,), idx, jnp.int32)
  w3tp = jnp.pad(W3_t, ((0, 0), (0, 16 - W3_t.shape[1])))
  b3tp = jnp.pad(b3_t, (0, 16 - b3_t.shape[0]))
  w3rp = jnp.pad(W3_r, ((0, 0), (0, 16 - W3_r.shape[1])))
  b3rp = jnp.pad(b3_r, (0, 16 - b3_r.shape[0]))
  ci = jnp.asarray(_CI)
  cf = jnp.asarray(_CF)
  iidx = jnp.asarray(_IIDX)
  tb = t_buf.reshape(-1)
  rb = r_buf.reshape(-1)

  mesh = plsc.VectorSubcoreMesh(core_axis_name="c", subcore_axis_name="s")
  f = pl.kernel(
      _body,
      out_type=(
          jax.ShapeDtypeStruct((16,), jnp.float32),
          jax.ShapeDtypeStruct((TW,), jnp.float32),
          jax.ShapeDtypeStruct((RW,), jnp.float32),
      ),
      mesh=mesh,
      compiler_params=pltpu.CompilerParams(needs_layout_passes=False),
      scratch_types=[
          pltpu.VMEM((16,), jnp.int32),
          pltpu.VMEM((8, 16), jnp.int32),
          pltpu.VMEM((4, 16), jnp.float32),
          pltpu.VMEM((2, 128), jnp.int32),
          pltpu.VMEM((16, E), jnp.float32),
          pltpu.VMEM((L, E), jnp.float32),
          pltpu.VMEM((L, E), jnp.float32),
          pltpu.VMEM((L, L), jnp.float32),
          pltpu.VMEM((E,), jnp.float32),
          pltpu.VMEM((L,), jnp.float32),
          pltpu.VMEM((2 * L,), jnp.float32),
          pltpu.VMEM((E,), jnp.float32),
          pltpu.VMEM((L,), jnp.float32),
          pltpu.VMEM((L,), jnp.float32),
          pltpu.VMEM((16,), jnp.int32),
          pltpu.VMEM((16,), jnp.float32),
          pltpu.VMEM((RCH,), jnp.float32),
          pltpu.VMEM_SHARED((E,), jnp.float32),
          pltpu.VMEM_SHARED((E,), jnp.float32),
          pltpu.VMEM_SHARED((L,), jnp.float32),
          pltpu.VMEM_SHARED((E,), jnp.float32),
          pltpu.VMEM_SHARED((E,), jnp.float32),
          pltpu.VMEM_SHARED((L,), jnp.float32),
          pltpu.SemaphoreType.DMA,
          pltpu.SemaphoreType.DMA,
      ],
  )
  c2w16, tbn, rbn = f(idx16, emb_t, W1_t, b1_t, W2_t, b2_t, w3tp, b3tp,
                      emb_r, W1_r, b1_r, W2_r, b2_r, w3rp, b3rp,
                      tb, rb, ci, cf, iidx)
  return (c2w16.reshape(4, 4), tbn.reshape(n, 3), rbn.reshape(n, 4))
